# baseline (speedup 1.0000x reference)
import functools

import jax
import jax.numpy as jnp
from jax import lax
from jax.experimental import pallas as pl
from jax.experimental.pallas import tpu as pltpu
from jax.experimental.pallas import tpu_sc as plsc

D = 64
NI = 4096
NJ = 200
B = NI * NJ
NW = 32
B_PER_W = B // NW
C = 512
NB = 2
NCH = B_PER_W // C
NG = NCH // NB


@functools.partial(
    pl.kernel,
    mesh=plsc.VectorSubcoreMesh(core_axis_name="c", subcore_axis_name="s"),
    out_type=jax.ShapeDtypeStruct((NI, NJ, D), jnp.float32),
    compiler_params=pltpu.CompilerParams(use_tc_tiling_on_sc=False),
    scratch_types=[
        pltpu.VMEM((C,), jnp.int32),
        pltpu.VMEM((C,), jnp.int32),
        pltpu.VMEM((C, D), jnp.float32),
        pltpu.VMEM((C, D), jnp.float32),
        pltpu.SemaphoreType.DMA,
        pltpu.SemaphoreType.DMA,
        pltpu.SemaphoreType.DMA,
        pltpu.SemaphoreType.DMA,
        pltpu.SemaphoreType.DMA,
        pltpu.SemaphoreType.DMA,
    ],
)
def _gather_kernel(item_t_hbm, table_hbm, out_hbm,
                   idx0, idx1, rows0, rows1,
                   si0, si1, sg0, sg1, so0, so1):
    wid = lax.axis_index("s") * 2 + lax.axis_index("c")
    base = wid * B_PER_W
    idxs = (idx0, idx1)
    rows = (rows0, rows1)
    sis = (si0, si1)
    sgs = (sg0, sg1)
    sos = (so0, so1)

    def jo(c):
        k = base + c * C
        return k // NI, k % NI

    def idx_start(c, b):
        j, i0 = jo(c)
        return pltpu.make_async_copy(
            item_t_hbm.at[j, pl.ds(i0, C)], idxs[b], sis[b])

    def out_copy(c, b):
        j, i0 = jo(c)
        return pltpu.make_async_copy(
            rows[b], out_hbm.at[pl.ds(i0, C), j, :], sos[b])

    def step(c, b, wait_prev_out, prefetch_next):
        idx_start(c, b).wait()
        if wait_prev_out:
            out_copy(c - NB, b).wait()
        pltpu.async_copy(table_hbm.at[idxs[b]], rows[b], sgs[b]).wait()
        if prefetch_next:
            idx_start(c + NB, b).start()
        out_copy(c, b).start()

    for b in range(NB):
        idx_start(b, b).start()

    for b in range(NB):
        step(b, b, wait_prev_out=False, prefetch_next=True)

    def outer(g, carry):
        for b in range(NB):
            step(g * NB + b, b, wait_prev_out=True, prefetch_next=True)
        return carry

    lax.fori_loop(1, NG - 1, outer, 0)

    for b in range(NB):
        step((NG - 1) * NB + b, b, wait_prev_out=True, prefetch_next=False)

    for b in range(NB):
        out_copy(NCH - NB + b, b).wait()


def kernel(item, table):
    return _gather_kernel(item.T, table)
